# baseline (speedup 1.0000x reference)
import functools

import jax
import jax.numpy as jnp
from jax import lax
from jax.experimental import pallas as pl
from jax.experimental.pallas import tpu as pltpu
from jax.experimental.pallas import tpu_sc as plsc

N = 10000
E = 320000
F_IN = 128
HID = 64
OUT = 300

NC = 2
NS = 16
NW = NC * NS
EPW = E // NW
CHUNK = 80
NCHUNK = EPW // CHUNK
NP = 10240
RPT = NP // NS
ZR = 128
DEGW = 16


def _proj_body(x_ref, wlt_ref, xl_ref):
    xl_ref[...] = jnp.dot(x_ref[...], wlt_ref[...],
                          preferred_element_type=jnp.float32)


def _proj(x, wlt):
    rb = 1000
    return pl.pallas_call(
        _proj_body,
        grid=(N // rb,),
        in_specs=[
            pl.BlockSpec((rb, F_IN), lambda i: (i, 0)),
            pl.BlockSpec((F_IN, HID), lambda i: (0, 0)),
        ],
        out_specs=pl.BlockSpec((rb, HID), lambda i: (i, 0)),
        out_shape=jax.ShapeDtypeStruct((N, HID), jnp.float32),
    )(x, wlt)


NSLOT = 5
ROUNDS = NCHUNK // NSLOT
LRPT = N - (NS - 1) * RPT
DEG_SLAB = 2000


def _sc_aggregate(xl, ei4):
    mesh = plsc.VectorSubcoreMesh(core_axis_name="c", subcore_axis_name="s")

    @functools.partial(
        pl.kernel,
        mesh=mesh,
        out_type=[
            jax.ShapeDtypeStruct((NC * N * HID // 128, 128), jnp.float32),
            jax.ShapeDtypeStruct((NC * DEG_SLAB, 128), jnp.float32),
        ],
        scratch_types=[
            pltpu.VMEM_SHARED((NP, HID), jnp.float32),
            pltpu.VMEM_SHARED((NP, DEGW), jnp.float32),
            pltpu.VMEM((NCHUNK, CHUNK), jnp.int32),
            pltpu.VMEM((NCHUNK, CHUNK), jnp.int32),
            pltpu.VMEM((NSLOT, 2, CHUNK, HID), jnp.float32),
            pltpu.VMEM((CHUNK, DEGW), jnp.float32),
            pltpu.VMEM((CHUNK, DEGW), jnp.float32),
            pltpu.VMEM((40, 128), jnp.float32),
            pltpu.SemaphoreType.DMA((NSLOT, 2)),
            pltpu.SemaphoreType.DMA((NSLOT, 2)),
            pltpu.SemaphoreType.DMA((NSLOT, 2)),
        ],
        compiler_params=pltpu.CompilerParams(use_tc_tiling_on_sc=False),
    )
    def sc_kernel(xl_hbm, ei_hbm, agg_out, deg_out,
                  agg_sh, deg_sh, src_all, dst_all, rows_v, ones_v,
                  zdeg_v, pk_v, sem_g, sem_s, sem_d):
        cid = lax.axis_index("c")
        sid = lax.axis_index("s")

        z16 = jnp.zeros((16,), jnp.float32)
        o16 = jnp.ones((16,), jnp.float32)

        zrow_v = rows_v.at[0, 0]

        def fill_bufs(i, carry):
            for j in range(HID // 16):
                zrow_v[i, pl.ds(j * 16, 16)] = z16
            zdeg_v[i, :] = z16
            ones_v[i, :] = o16
            return carry

        lax.fori_loop(0, CHUNK, fill_bufs, 0)

        rbase = sid * RPT
        for k in range(RPT // CHUNK):
            pltpu.sync_copy(zrow_v, agg_sh.at[pl.ds(rbase + k * CHUNK, CHUNK)])
            pltpu.sync_copy(zdeg_v, deg_sh.at[pl.ds(rbase + k * CHUNK, CHUNK)])
        wid = sid * NC + cid
        pltpu.sync_copy(ei_hbm.at[0, wid], src_all)
        pltpu.sync_copy(ei_hbm.at[1, wid], dst_all)
        plsc.subcore_barrier()

        def gather(c, k, b):
            pltpu.async_copy(xl_hbm.at[src_all.at[c]], rows_v.at[k, b],
                             sem_g.at[k, b])

        def gather_wait(c, k, b):
            pltpu.make_async_copy(xl_hbm.at[src_all.at[c]], rows_v.at[k, b],
                                  sem_g.at[k, b]).wait()

        def scat(c, k, b):
            pltpu.async_copy(rows_v.at[k, b], agg_sh.at[dst_all.at[c]],
                             sem_s.at[k, b], add=True)
            pltpu.async_copy(ones_v, deg_sh.at[dst_all.at[c]],
                             sem_d.at[k, b], add=True)

        def scat_wait(c, k, b):
            pltpu.make_async_copy(rows_v.at[k, b], agg_sh.at[dst_all.at[c]],
                                  sem_s.at[k, b]).wait()
            pltpu.make_async_copy(ones_v, deg_sh.at[dst_all.at[c]],
                                  sem_d.at[k, b]).wait()

        for k in range(NSLOT):
            gather(k, k, 0)

        def round_body(r, b):
            nb = 1 - b
            for k in range(NSLOT):
                c = r * NSLOT + k
                pc = c + NSLOT

                @pl.when(pc < NCHUNK)
                def _():
                    @pl.when(r > 0)
                    def _():
                        scat_wait(c - NSLOT, k, nb)
                    gather(pc, k, nb)

                gather_wait(c, k, b)
                scat(c, k, b)

        def two_rounds(j2, carry):
            round_body(2 * j2, 0)
            round_body(2 * j2 + 1, 1)
            return carry

        lax.fori_loop(0, (ROUNDS - 1) // 2, two_rounds, 0)

        for k in range(NSLOT):
            c = (ROUNDS - 1) * NSLOT + k
            gather_wait(c, k, 0)
            scat(c, k, 0)
        for k in range(NSLOT):
            scat_wait((ROUNDS - 2) * NSLOT + k, k, 1)
            scat_wait((ROUNDS - 1) * NSLOT + k, k, 0)
        plsc.subcore_barrier()

        pbase = (cid * N + rbase) * HID // 128
        dbase = cid * DEG_SLAB + rbase * DEGW // 128

        def drain_agg(nck):
            for k in range(nck):
                a = rows_v.at[k % NSLOT, k // NSLOT]
                pltpu.sync_copy(agg_sh.at[pl.ds(rbase + k * CHUNK, CHUNK)], a)

                def prow(r, carry):
                    for j in range(8):
                        pk_v[r, pl.ds(j * 16, 16)] = (
                            a[2 * r + j // 4, pl.ds((j % 4) * 16, 16)])
                    return carry

                lax.fori_loop(0, CHUNK // 2, prow, 0)
                pltpu.sync_copy(
                    pk_v, agg_out.at[pl.ds(pbase + k * (CHUNK // 2),
                                           CHUNK // 2)])

        def drain_deg(nck):
            for m in range((nck + 3) // 4):
                for q in range(min(4, nck - 4 * m)):
                    pltpu.sync_copy(
                        deg_sh.at[pl.ds(rbase + (4 * m + q) * CHUNK, CHUNK)],
                        zdeg_v)

                    def drow(p, carry, _q=q):
                        for j in range(8):
                            pk_v[_q * 10 + p, pl.ds(j * 16, 16)] = (
                                zdeg_v[8 * p + j, :])
                        return carry

                    lax.fori_loop(0, 10, drow, 0)
                nrow = 40 if 4 * (m + 1) <= nck else 16
                pltpu.sync_copy(
                    pk_v.at[pl.ds(0, nrow)],
                    deg_out.at[pl.ds(dbase + m * 40, nrow)])

        @pl.when(sid < NS - 1)
        def _():
            drain_agg(RPT // CHUNK)
            drain_deg(RPT // CHUNK)

        @pl.when(sid == NS - 1)
        def _():
            drain_agg(LRPT // CHUNK)
            drain_deg(LRPT // CHUNK)

    return sc_kernel(xl, ei4)


def _recip_body(deg_ref, r64_ref):
    dr = N * DEGW // 128
    dp = deg_ref[0:dr] + deg_ref[DEG_SLAB:DEG_SLAB + dr]
    sel = (lax.broadcasted_iota(jnp.int32, (128, 8), 0) // DEGW
           == lax.broadcasted_iota(jnp.int32, (128, 8), 1)
           ).astype(jnp.float32)
    deg = jnp.dot(dp, sel, preferred_element_type=jnp.float32) * (1.0 / DEGW)
    recip = 1.0 / jnp.maximum(deg, 1.0)
    r64_ref[...] = jnp.broadcast_to(recip[:, :, None],
                                    (dr, 8, HID)).reshape(N, HID)


def _recip(deg_flat):
    return pl.pallas_call(
        _recip_body,
        out_shape=jax.ShapeDtypeStruct((N, HID), jnp.float32),
    )(deg_flat)


def _head_body(agg0_ref, agg1_ref, r64_ref, x_ref, wrt_ref, bl_ref, w2t_ref,
               b2_ref, y_ref):
    rb = r64_ref.shape[0]
    ap = agg0_ref[...] + agg1_ref[...]
    lo = ap[:, 0:HID][:, None, :]
    hi = ap[:, HID:128][:, None, :]
    agg = jnp.concatenate([lo, hi], axis=1).reshape(rb, HID)
    xr = jnp.dot(x_ref[...], wrt_ref[...], preferred_element_type=jnp.float32)
    h = agg * r64_ref[...] + bl_ref[...] + xr
    h = jnp.maximum(h, 0.0)
    y = (jnp.dot(h, w2t_ref[...], preferred_element_type=jnp.float32)
         + b2_ref[...])
    y_ref[...] = jnp.concatenate(
        [y[:, 0:100][:, None, :], y[:, 100:200][:, None, :],
         y[:, 200:300][:, None, :]], axis=1).reshape(3 * rb, 100)


def _head(agg_flat, r64, x, wrt, bl, w2t, b2):
    rb = 2000
    nb = N // rb
    pb = rb * HID // 128
    return pl.pallas_call(
        _head_body,
        grid=(nb,),
        in_specs=[
            pl.BlockSpec((pb, 128), lambda i: (i, 0)),
            pl.BlockSpec((pb, 128), lambda i: (i + nb, 0)),
            pl.BlockSpec((rb, HID), lambda i: (i, 0)),
            pl.BlockSpec((rb, F_IN), lambda i: (i, 0)),
            pl.BlockSpec((F_IN, HID), lambda i: (0, 0)),
            pl.BlockSpec((1, HID), lambda i: (0, 0)),
            pl.BlockSpec((HID, OUT), lambda i: (0, 0)),
            pl.BlockSpec((1, OUT), lambda i: (0, 0)),
        ],
        out_specs=pl.BlockSpec((3 * rb, 100), lambda i: (i, 0)),
        out_shape=jax.ShapeDtypeStruct((3 * N, 100), jnp.float32),
    )(agg_flat, agg_flat, r64, x, wrt, bl, w2t, b2)


def kernel(x, edge_index, batch, Wl, bl, Wr, W2, b2):
    ei4 = edge_index.reshape(2, NW, NCHUNK, CHUNK)
    xl = _proj(x, Wl.T)
    agg_flat, deg_flat = _sc_aggregate(xl, ei4)
    r64 = _recip(deg_flat)
    return _head(agg_flat, r64, x, Wr.T, bl.reshape(1, HID), W2.T,
                 b2.reshape(1, OUT))


# speedup vs baseline: 1.0123x; 1.0123x over previous
import functools

import jax
import jax.numpy as jnp
from jax import lax
from jax.experimental import pallas as pl
from jax.experimental.pallas import tpu as pltpu
from jax.experimental.pallas import tpu_sc as plsc

N = 10000
E = 320000
F_IN = 128
HID = 64
OUT = 300

NC = 2
NS = 16
NW = NC * NS
EPW = E // NW
CHUNK = 80
NCHUNK = EPW // CHUNK
NP = 10240
RPT = NP // NS
ZR = 128
DEGW = 16


def _proj_body(x_ref, wlt_ref, xl_ref):
    xl_ref[...] = jnp.dot(x_ref[...], wlt_ref[...],
                          preferred_element_type=jnp.float32)


def _proj(x, wlt):
    rb = 1000
    return pl.pallas_call(
        _proj_body,
        grid=(N // rb,),
        in_specs=[
            pl.BlockSpec((rb, F_IN), lambda i: (i, 0)),
            pl.BlockSpec((F_IN, HID), lambda i: (0, 0)),
        ],
        out_specs=pl.BlockSpec((rb, HID), lambda i: (i, 0)),
        out_shape=jax.ShapeDtypeStruct((N, HID), jnp.float32),
    )(x, wlt)


NSLOT = 5
ROUNDS = NCHUNK // NSLOT
LRPT = N - (NS - 1) * RPT
DEG_SLAB = 2000


def _sc_aggregate(xl, ei4):
    mesh = plsc.VectorSubcoreMesh(core_axis_name="c", subcore_axis_name="s")

    @functools.partial(
        pl.kernel,
        mesh=mesh,
        out_type=[
            jax.ShapeDtypeStruct((NC * N * HID // 128, 128), jnp.float32),
            jax.ShapeDtypeStruct((NC * DEG_SLAB, 128), jnp.float32),
        ],
        scratch_types=[
            pltpu.VMEM_SHARED((NP, HID), jnp.float32),
            pltpu.VMEM_SHARED((NP, DEGW), jnp.float32),
            pltpu.VMEM((NCHUNK, CHUNK), jnp.int32),
            pltpu.VMEM((NCHUNK, CHUNK), jnp.int32),
            pltpu.VMEM((NSLOT, 2, CHUNK, HID), jnp.float32),
            pltpu.VMEM((CHUNK, DEGW), jnp.float32),
            pltpu.VMEM((CHUNK, DEGW), jnp.float32),
            pltpu.VMEM((40, 128), jnp.float32),
            pltpu.SemaphoreType.DMA((NSLOT, 2)),
            pltpu.SemaphoreType.DMA((NSLOT, 2)),
            pltpu.SemaphoreType.DMA((NSLOT, 2)),
        ],
        compiler_params=pltpu.CompilerParams(use_tc_tiling_on_sc=False),
    )
    def sc_kernel(xl_hbm, ei_hbm, agg_out, deg_out,
                  agg_sh, deg_sh, src_all, dst_all, rows_v, ones_v,
                  zdeg_v, pk_v, sem_g, sem_s, sem_d):
        cid = lax.axis_index("c")
        sid = lax.axis_index("s")

        z16 = jnp.zeros((16,), jnp.float32)
        o16 = jnp.ones((16,), jnp.float32)

        zrow_v = rows_v.at[0, 0]

        def fill_bufs(i, carry):
            for j in range(HID // 16):
                zrow_v[i, pl.ds(j * 16, 16)] = z16
            zdeg_v[i, :] = z16
            ones_v[i, :] = o16
            return carry

        lax.fori_loop(0, CHUNK, fill_bufs, 0)

        rbase = sid * RPT
        for k in range(RPT // CHUNK):
            pltpu.sync_copy(zrow_v, agg_sh.at[pl.ds(rbase + k * CHUNK, CHUNK)])
            pltpu.sync_copy(zdeg_v, deg_sh.at[pl.ds(rbase + k * CHUNK, CHUNK)])
        wid = sid * NC + cid
        pltpu.sync_copy(ei_hbm.at[0, wid], src_all)
        pltpu.sync_copy(ei_hbm.at[1, wid], dst_all)
        plsc.subcore_barrier()

        def gather(c, k, b):
            pltpu.async_copy(xl_hbm.at[src_all.at[c]], rows_v.at[k, b],
                             sem_g.at[k, b])

        def gather_wait(c, k, b):
            pltpu.make_async_copy(xl_hbm.at[src_all.at[c]], rows_v.at[k, b],
                                  sem_g.at[k, b]).wait()

        def scat(c, k, b):
            pltpu.async_copy(rows_v.at[k, b], agg_sh.at[dst_all.at[c]],
                             sem_s.at[k, b], add=True)
            pltpu.async_copy(ones_v, deg_sh.at[dst_all.at[c]],
                             sem_d.at[k, b], add=True)

        def scat_wait(c, k, b):
            pltpu.make_async_copy(rows_v.at[k, b], agg_sh.at[dst_all.at[c]],
                                  sem_s.at[k, b]).wait()
            pltpu.make_async_copy(ones_v, deg_sh.at[dst_all.at[c]],
                                  sem_d.at[k, b]).wait()

        for k in range(NSLOT):
            gather(k, k, 0)

        def round_body(r, b):
            nb = 1 - b
            for k in range(NSLOT):
                c = r * NSLOT + k
                pc = c + NSLOT

                @pl.when(pc < NCHUNK)
                def _():
                    @pl.when(r > 0)
                    def _():
                        scat_wait(c - NSLOT, k, nb)
                    gather(pc, k, nb)

                gather_wait(c, k, b)
                scat(c, k, b)

        def two_rounds(j2, carry):
            round_body(2 * j2, 0)
            round_body(2 * j2 + 1, 1)
            return carry

        lax.fori_loop(0, (ROUNDS - 1) // 2, two_rounds, 0)

        for k in range(NSLOT):
            c = (ROUNDS - 1) * NSLOT + k
            gather_wait(c, k, 0)
            scat(c, k, 0)
        for k in range(NSLOT):
            scat_wait((ROUNDS - 2) * NSLOT + k, k, 1)
            scat_wait((ROUNDS - 1) * NSLOT + k, k, 0)
        plsc.subcore_barrier()

        pbase = (cid * N + rbase) * HID // 128
        dbase = cid * DEG_SLAB + rbase * DEGW // 128

        def drain_agg(nck):
            for k in range(nck):
                a = rows_v.at[k % NSLOT, k // NSLOT]
                pltpu.sync_copy(agg_sh.at[pl.ds(rbase + k * CHUNK, CHUNK)], a)

                def prow(r, carry):
                    for j in range(8):
                        pk_v[r, pl.ds(j * 16, 16)] = (
                            a[2 * r + j // 4, pl.ds((j % 4) * 16, 16)])
                    return carry

                lax.fori_loop(0, CHUNK // 2, prow, 0)
                pltpu.sync_copy(
                    pk_v, agg_out.at[pl.ds(pbase + k * (CHUNK // 2),
                                           CHUNK // 2)])

        def drain_deg(nck):
            for m in range((nck + 3) // 4):
                for q in range(min(4, nck - 4 * m)):
                    pltpu.sync_copy(
                        deg_sh.at[pl.ds(rbase + (4 * m + q) * CHUNK, CHUNK)],
                        zdeg_v)

                    def drow(p, carry, _q=q):
                        for j in range(8):
                            pk_v[_q * 10 + p, pl.ds(j * 16, 16)] = (
                                zdeg_v[8 * p + j, :])
                        return carry

                    lax.fori_loop(0, 10, drow, 0)
                nrow = 40 if 4 * (m + 1) <= nck else 16
                pltpu.sync_copy(
                    pk_v.at[pl.ds(0, nrow)],
                    deg_out.at[pl.ds(dbase + m * 40, nrow)])

        @pl.when(sid < NS - 1)
        def _():
            drain_agg(RPT // CHUNK)
            drain_deg(RPT // CHUNK)

        @pl.when(sid == NS - 1)
        def _():
            drain_agg(LRPT // CHUNK)
            drain_deg(LRPT // CHUNK)

    return sc_kernel(xl, ei4)


def _recip_body(deg_ref, r64_ref):
    dr = N * DEGW // 128
    dp = deg_ref[0:dr] + deg_ref[DEG_SLAB:DEG_SLAB + dr]
    sel = (lax.broadcasted_iota(jnp.int32, (128, 8), 0) // DEGW
           == lax.broadcasted_iota(jnp.int32, (128, 8), 1)
           ).astype(jnp.float32)
    deg = jnp.dot(dp, sel, preferred_element_type=jnp.float32) * (1.0 / DEGW)
    recip = 1.0 / jnp.maximum(deg, 1.0)
    r64_ref[...] = jnp.broadcast_to(recip[:, :, None],
                                    (dr, 8, HID)).reshape(N, HID)


def _recip(deg_flat):
    return pl.pallas_call(
        _recip_body,
        out_shape=jax.ShapeDtypeStruct((N, HID), jnp.float32),
    )(deg_flat)


def _head_body(agg0_ref, agg1_ref, r64_ref, x_ref, wrt_ref, bl_ref, w2s_ref,
               b23_ref, y_ref):
    rb = r64_ref.shape[0]
    ap = agg0_ref[...] + agg1_ref[...]
    lo = ap[:, 0:HID][:, None, :]
    hi = ap[:, HID:128][:, None, :]
    agg = jnp.concatenate([lo, hi], axis=1).reshape(rb, HID)
    xr = jnp.dot(x_ref[...], wrt_ref[...], preferred_element_type=jnp.float32)
    h = agg * r64_ref[...] + bl_ref[...] + xr
    h = jnp.maximum(h, 0.0)
    z = jnp.zeros((rb, HID), jnp.float32)
    h3 = jnp.concatenate([
        jnp.concatenate([h, z, z], axis=1)[:, None, :],
        jnp.concatenate([z, h, z], axis=1)[:, None, :],
        jnp.concatenate([z, z, h], axis=1)[:, None, :]], axis=1)
    h3 = h3.reshape(3 * rb, 3 * HID)
    b3 = jnp.broadcast_to(b23_ref[...][None], (rb, 3, 100)).reshape(3 * rb,
                                                                    100)
    y_ref[...] = (jnp.dot(h3, w2s_ref[...], preferred_element_type=jnp.float32)
                  + b3)


def _head(agg_flat, r64, x, wrt, bl, w2s, b23):
    rb = 2000
    nb = N // rb
    pb = rb * HID // 128
    return pl.pallas_call(
        _head_body,
        grid=(nb,),
        in_specs=[
            pl.BlockSpec((pb, 128), lambda i: (i, 0)),
            pl.BlockSpec((pb, 128), lambda i: (i + nb, 0)),
            pl.BlockSpec((rb, HID), lambda i: (i, 0)),
            pl.BlockSpec((rb, F_IN), lambda i: (i, 0)),
            pl.BlockSpec((F_IN, HID), lambda i: (0, 0)),
            pl.BlockSpec((1, HID), lambda i: (0, 0)),
            pl.BlockSpec((3 * HID, 100), lambda i: (0, 0)),
            pl.BlockSpec((3, 100), lambda i: (0, 0)),
        ],
        out_specs=pl.BlockSpec((3 * rb, 100), lambda i: (i, 0)),
        out_shape=jax.ShapeDtypeStruct((3 * N, 100), jnp.float32),
    )(agg_flat, agg_flat, r64, x, wrt, bl, w2s, b23)


def kernel(x, edge_index, batch, Wl, bl, Wr, W2, b2):
    ei4 = edge_index.reshape(2, NW, NCHUNK, CHUNK)
    xl = _proj(x, Wl.T)
    agg_flat, deg_flat = _sc_aggregate(xl, ei4)
    r64 = _recip(deg_flat)
    w2t = W2.T
    w2s = jnp.concatenate([w2t[:, 0:100], w2t[:, 100:200], w2t[:, 200:300]],
                          axis=0)
    return _head(agg_flat, r64, x, Wr.T, bl.reshape(1, HID), w2s,
                 b2.reshape(3, 100))


# speedup vs baseline: 1.1127x; 1.0992x over previous
import functools

import jax
import jax.numpy as jnp
from jax import lax
from jax.experimental import pallas as pl
from jax.experimental.pallas import tpu as pltpu
from jax.experimental.pallas import tpu_sc as plsc

N = 10000
E = 320000
F_IN = 128
HID = 64
OUT = 300

NC = 2
NS = 16
NW = NC * NS
EPW = E // NW
CHUNK = 80
NCHUNK = EPW // CHUNK
NP = 10240
RPT = NP // NS
ZR = 128
DEGW = 16


def _proj_body(x_ref, wlt_ref, xl_ref):
    xl_ref[...] = jnp.dot(x_ref[...], wlt_ref[...],
                          preferred_element_type=jnp.float32)


def _proj(x, wlt):
    rb = 1000
    return pl.pallas_call(
        _proj_body,
        grid=(N // rb,),
        in_specs=[
            pl.BlockSpec((rb, F_IN), lambda i: (i, 0)),
            pl.BlockSpec((F_IN, HID), lambda i: (0, 0)),
        ],
        out_specs=pl.BlockSpec((rb, HID), lambda i: (i, 0)),
        out_shape=jax.ShapeDtypeStruct((N, HID), jnp.float32),
    )(x, wlt)


NSLOT = 5
ROUNDS = NCHUNK // NSLOT
LRPT = N - (NS - 1) * RPT
DEG_SLAB = 2000


def _sc_aggregate(xl, ei4):
    mesh = plsc.VectorSubcoreMesh(core_axis_name="c", subcore_axis_name="s")

    @functools.partial(
        pl.kernel,
        mesh=mesh,
        out_type=[
            jax.ShapeDtypeStruct((NC * N * HID // 128, 128), jnp.float32),
            jax.ShapeDtypeStruct((NC * DEG_SLAB, 128), jnp.float32),
        ],
        scratch_types=[
            pltpu.VMEM_SHARED((NP, HID), jnp.float32),
            pltpu.VMEM_SHARED((NP, DEGW), jnp.float32),
            pltpu.VMEM((NCHUNK, CHUNK), jnp.int32),
            pltpu.VMEM((NCHUNK, CHUNK), jnp.int32),
            pltpu.VMEM((NSLOT, 2, CHUNK, HID), jnp.float32),
            pltpu.VMEM((CHUNK, DEGW), jnp.float32),
            pltpu.VMEM((CHUNK, DEGW), jnp.float32),
            pltpu.VMEM((40, 128), jnp.float32),
            pltpu.SemaphoreType.DMA((NSLOT, 2)),
            pltpu.SemaphoreType.DMA((NSLOT, 2)),
            pltpu.SemaphoreType.DMA((NSLOT, 2)),
        ],
        compiler_params=pltpu.CompilerParams(use_tc_tiling_on_sc=False),
    )
    def sc_kernel(xl_hbm, ei_hbm, agg_out, deg_out,
                  agg_sh, deg_sh, src_all, dst_all, rows_v, ones_v,
                  zdeg_v, pk_v, sem_g, sem_s, sem_d):
        cid = lax.axis_index("c")
        sid = lax.axis_index("s")

        z16 = jnp.zeros((16,), jnp.float32)
        o16 = jnp.ones((16,), jnp.float32)

        zrow_v = rows_v.at[0, 0]

        def fill_bufs(i, carry):
            for j in range(HID // 16):
                zrow_v[i, pl.ds(j * 16, 16)] = z16
            zdeg_v[i, :] = z16
            ones_v[i, :] = o16
            return carry

        lax.fori_loop(0, CHUNK, fill_bufs, 0)

        rbase = sid * RPT
        for k in range(RPT // CHUNK):
            pltpu.sync_copy(zrow_v, agg_sh.at[pl.ds(rbase + k * CHUNK, CHUNK)])
            pltpu.sync_copy(zdeg_v, deg_sh.at[pl.ds(rbase + k * CHUNK, CHUNK)])
        wid = sid * NC + cid
        pltpu.sync_copy(ei_hbm.at[0, wid], src_all)
        pltpu.sync_copy(ei_hbm.at[1, wid], dst_all)
        plsc.subcore_barrier()

        def gather(c, k, b):
            pltpu.async_copy(xl_hbm.at[src_all.at[c]], rows_v.at[k, b],
                             sem_g.at[k, b])

        def gather_wait(c, k, b):
            pltpu.make_async_copy(xl_hbm.at[src_all.at[c]], rows_v.at[k, b],
                                  sem_g.at[k, b]).wait()

        def scat(c, k, b):
            pltpu.async_copy(rows_v.at[k, b], agg_sh.at[dst_all.at[c]],
                             sem_s.at[k, b], add=True)
            pltpu.async_copy(ones_v, deg_sh.at[dst_all.at[c]],
                             sem_d.at[k, b], add=True)

        def scat_wait(c, k, b):
            pltpu.make_async_copy(rows_v.at[k, b], agg_sh.at[dst_all.at[c]],
                                  sem_s.at[k, b]).wait()
            pltpu.make_async_copy(ones_v, deg_sh.at[dst_all.at[c]],
                                  sem_d.at[k, b]).wait()

        for k in range(NSLOT):
            gather(k, k, 0)

        def round_body(r, b):
            nb = 1 - b
            for k in range(NSLOT):
                c = r * NSLOT + k
                pc = c + NSLOT

                @pl.when(pc < NCHUNK)
                def _():
                    @pl.when(r > 0)
                    def _():
                        scat_wait(c - NSLOT, k, nb)
                    gather(pc, k, nb)

                gather_wait(c, k, b)
                scat(c, k, b)

        def two_rounds(j2, carry):
            round_body(2 * j2, 0)
            round_body(2 * j2 + 1, 1)
            return carry

        lax.fori_loop(0, (ROUNDS - 1) // 2, two_rounds, 0)

        for k in range(NSLOT):
            c = (ROUNDS - 1) * NSLOT + k
            gather_wait(c, k, 0)
            scat(c, k, 0)
        for k in range(NSLOT):
            scat_wait((ROUNDS - 2) * NSLOT + k, k, 1)
            scat_wait((ROUNDS - 1) * NSLOT + k, k, 0)
        plsc.subcore_barrier()

        pbase = (cid * N + rbase) * HID // 128
        dbase = cid * DEG_SLAB + rbase * DEGW // 128

        def drain_agg(nck):
            for k in range(nck):
                a = rows_v.at[k % NSLOT, k // NSLOT]
                pltpu.sync_copy(agg_sh.at[pl.ds(rbase + k * CHUNK, CHUNK)], a)

                def prow(r, carry):
                    for j in range(8):
                        pk_v[r, pl.ds(j * 16, 16)] = (
                            a[2 * r + j // 4, pl.ds((j % 4) * 16, 16)])
                    return carry

                lax.fori_loop(0, CHUNK // 2, prow, 0)
                pltpu.sync_copy(
                    pk_v, agg_out.at[pl.ds(pbase + k * (CHUNK // 2),
                                           CHUNK // 2)])

        def drain_deg(nck):
            for m in range((nck + 3) // 4):
                for q in range(min(4, nck - 4 * m)):
                    pltpu.sync_copy(
                        deg_sh.at[pl.ds(rbase + (4 * m + q) * CHUNK, CHUNK)],
                        zdeg_v)

                    def drow(p, carry, _q=q):
                        for j in range(8):
                            pk_v[_q * 10 + p, pl.ds(j * 16, 16)] = (
                                zdeg_v[8 * p + j, :])
                        return carry

                    lax.fori_loop(0, 10, drow, 0)
                nrow = 40 if 4 * (m + 1) <= nck else 16
                pltpu.sync_copy(
                    pk_v.at[pl.ds(0, nrow)],
                    deg_out.at[pl.ds(dbase + m * 40, nrow)])

        @pl.when(sid < NS - 1)
        def _():
            drain_agg(RPT // CHUNK)
            drain_deg(RPT // CHUNK)

        @pl.when(sid == NS - 1)
        def _():
            drain_agg(LRPT // CHUNK)
            drain_deg(LRPT // CHUNK)

    return sc_kernel(xl, ei4)


def _recip_body(deg_ref, r64_ref):
    dr = N * DEGW // 128
    dp = deg_ref[0:dr] + deg_ref[DEG_SLAB:DEG_SLAB + dr]
    sel = (lax.broadcasted_iota(jnp.int32, (128, 8), 0) // DEGW
           == lax.broadcasted_iota(jnp.int32, (128, 8), 1)
           ).astype(jnp.float32)
    deg = jnp.dot(dp, sel, preferred_element_type=jnp.float32) * (1.0 / DEGW)
    recip = 1.0 / jnp.maximum(deg, 1.0)
    r64_ref[...] = jnp.broadcast_to(recip[:, :, None],
                                    (dr, 8, HID)).reshape(N, HID)


def _recip(deg_flat):
    return pl.pallas_call(
        _recip_body,
        out_shape=jax.ShapeDtypeStruct((N, HID), jnp.float32),
    )(deg_flat)


def _head_body(agg0_ref, agg1_ref, r64_ref, x_ref, wrt_ref, bl_ref, w2s_ref,
               b23_ref, y_ref):
    rb = r64_ref.shape[0]
    ap = agg0_ref[...] + agg1_ref[...]
    lo = ap[:, 0:HID][:, None, :]
    hi = ap[:, HID:128][:, None, :]
    agg = jnp.concatenate([lo, hi], axis=1).reshape(rb, HID)
    xr = jnp.dot(x_ref[...], wrt_ref[...], preferred_element_type=jnp.float32)
    h = agg * r64_ref[...] + bl_ref[...] + xr
    h = jnp.maximum(h, 0.0)
    y_ref[...] = (jnp.dot(h, w2s_ref[...], preferred_element_type=jnp.float32)
                  + b23_ref[...])


def _head(agg_flat, r64, x, wrt, bl, w2t, b2):
    rb = 2000
    nb = N // rb
    pb = rb * HID // 128
    return pl.pallas_call(
        _head_body,
        grid=(nb,),
        in_specs=[
            pl.BlockSpec((pb, 128), lambda i: (i, 0)),
            pl.BlockSpec((pb, 128), lambda i: (i + nb, 0)),
            pl.BlockSpec((rb, HID), lambda i: (i, 0)),
            pl.BlockSpec((rb, F_IN), lambda i: (i, 0)),
            pl.BlockSpec((F_IN, HID), lambda i: (0, 0)),
            pl.BlockSpec((1, HID), lambda i: (0, 0)),
            pl.BlockSpec((HID, OUT), lambda i: (0, 0)),
            pl.BlockSpec((1, OUT), lambda i: (0, 0)),
        ],
        out_specs=pl.BlockSpec((rb, OUT), lambda i: (i, 0)),
        out_shape=jax.ShapeDtypeStruct((N, OUT), jnp.float32),
    )(agg_flat, agg_flat, r64, x, wrt, bl, w2t, b2)


def kernel(x, edge_index, batch, Wl, bl, Wr, W2, b2):
    ei4 = edge_index.reshape(2, NW, NCHUNK, CHUNK)
    xl = _proj(x, Wl.T)
    agg_flat, deg_flat = _sc_aggregate(xl, ei4)
    r64 = _recip(deg_flat)
    y = _head(agg_flat, r64, x, Wr.T, bl.reshape(1, HID), W2.T,
              b2.reshape(1, OUT))
    return y.reshape(-1, 100)
